# baseline (speedup 1.0000x reference)
import functools

import jax
import jax.numpy as jnp
from jax.experimental import pallas as pl
from jax.experimental.pallas import tpu as pltpu
from jax.experimental.pallas import tpu_sc as plsc

_B, _CIN, _T = 16, 128, 2048
_H, _D, _K = 384, 256, 1024
_S = 512
_BF = jnp.bfloat16

_NC, _NS, _L = 2, 16, 16
_NW = _NC * _NS
_DPW = _D // _NW


def _sd(m):
    z = jnp.zeros((1, m.shape[1]), m.dtype)
    return jnp.concatenate([z, m[:-1]], axis=0)


def _su(m):
    z = jnp.zeros((1, m.shape[1]), m.dtype)
    return jnp.concatenate([m[1:], z], axis=0)


def _mm(a, b):
    return jax.lax.dot_general(
        a, b, (((1,), (0,)), ((), ())),
        preferred_element_type=jnp.float32,
        precision=jax.lax.Precision.DEFAULT)


def _mm_tt(a, b):
    return jax.lax.dot_general(
        a, b, (((1,), (1,)), ((), ())),
        preferred_element_type=jnp.float32,
        precision=jax.lax.Precision.DEFAULT)


def _tc_body(x_ref, w12_ref, b1_ref, v12_ref,
             b2_ref, wz_ref, b3_ref, cbf_ref, cbh_ref,
             codes_ref, loss_ref):
    b = pl.program_id(0)
    xb = x_ref[0].astype(_BF)
    xt = xb.T
    xg = xt.reshape(1024, 256)

    xcat = jnp.concatenate(
        [_sd(xg[:, _CIN:]), xg, _su(xg[:, :_CIN])], axis=1)
    h1 = _mm(xcat, w12_ref[...]) + b1_ref[...]
    h1 = jnp.maximum(h1, 0.0).astype(_BF)

    hg = h1.reshape(512, 768)
    hcat = jnp.concatenate(
        [_sd(hg[:, _H:]), hg, _su(hg[:, :_H])], axis=1)
    h2 = _mm(hcat, v12_ref[...]) + b2_ref[...]
    h2 = jnp.maximum(h2, 0.0).astype(_BF)

    z = _mm(h2, wz_ref[...]) + b3_ref[...]

    cbf = cbf_ref[...]
    g = _mm_tt(z.astype(_BF), cbh_ref[...])
    znorm = jnp.sum(z * z, axis=1, keepdims=True)
    cnorm = jnp.sum(cbf * cbf, axis=1, keepdims=True)
    d = (znorm - 2.0 * g) + cnorm.reshape(1, _K)
    m = jnp.min(d, axis=1, keepdims=True)
    iota = jax.lax.broadcasted_iota(jnp.int32, (_S, _K), 1)
    codes_ref[0] = jnp.min(jnp.where(d == m, iota, _K), axis=1,
                           keepdims=True)

    part = jnp.sum(m)

    @pl.when(b == 0)
    def _():
        loss_ref[...] = jnp.zeros_like(loss_ref)

    loss_ref[...] += part


def _sc_body(cbt_hbm, codes_hbm, out_hbm, codes_v, cb_v, out_v):
    wid = jax.lax.axis_index("s") * _NC + jax.lax.axis_index("c")
    d0 = wid * _DPW
    pltpu.sync_copy(codes_hbm, codes_v)
    pltpu.sync_copy(cbt_hbm.at[pl.ds(d0 * _K, _DPW * _K)], cb_v)

    def body_b(b, carry):
        def body_d(d, carry):
            base = jnp.full((_L,), d * _K, jnp.int32)
            for j in range(_S // _L):
                idx = codes_v[b, pl.ds(j * _L, _L)] + base
                out_v[b, d, pl.ds(j * _L, _L)] = plsc.load_gather(cb_v, [idx])
            return carry
        return jax.lax.fori_loop(0, _DPW, body_d, carry)

    jax.lax.fori_loop(0, _B, body_b, 0)
    pltpu.sync_copy(out_v, out_hbm.at[:, pl.ds(d0, _DPW), :])


def _sc_lookup(cbt, codes):
    return functools.partial(
        pl.kernel,
        mesh=plsc.VectorSubcoreMesh(core_axis_name="c", subcore_axis_name="s"),
        compiler_params=pltpu.CompilerParams(needs_layout_passes=False),
        out_type=jax.ShapeDtypeStruct((_B, _D, _S), jnp.float32),
        scratch_types=[
            pltpu.VMEM((_B, _S), jnp.int32),
            pltpu.VMEM((_DPW * _K,), jnp.float32),
            pltpu.VMEM((_B, _DPW, _S), jnp.float32),
        ],
    )(_sc_body)(cbt, codes)


def kernel(input, W1, b1, W2, b2, W3, b3, codebook):
    w1 = W1.transpose(2, 1, 0).astype(_BF)
    w12 = w1.reshape(4 * _CIN, _H)
    v = W2.transpose(2, 1, 0).astype(_BF)
    v12 = v.reshape(4 * _H, _H)
    wz = W3[:, :, 0].T.astype(_BF)
    cb_hi = codebook.astype(_BF)
    cbt = codebook.T.reshape(-1)

    codes, loss = pl.pallas_call(
        _tc_body,
        grid=(_B,),
        in_specs=[
            pl.BlockSpec((1, _CIN, _T), lambda b: (b, 0, 0)),
            pl.BlockSpec((4 * _CIN, _H), lambda b: (0, 0)),
            pl.BlockSpec((1, _H), lambda b: (0, 0)),
            pl.BlockSpec((4 * _H, _H), lambda b: (0, 0)),
            pl.BlockSpec((1, _H), lambda b: (0, 0)),
            pl.BlockSpec((_H, _D), lambda b: (0, 0)),
            pl.BlockSpec((1, _D), lambda b: (0, 0)),
            pl.BlockSpec((_K, _D), lambda b: (0, 0)),
            pl.BlockSpec((_K, _D), lambda b: (0, 0)),
        ],
        out_specs=[
            pl.BlockSpec((1, _S, 1), lambda b: (b, 0, 0)),
            pl.BlockSpec((1, 1), lambda b: (0, 0)),
        ],
        out_shape=[
            jax.ShapeDtypeStruct((_B, _S, 1), jnp.int32),
            jax.ShapeDtypeStruct((1, 1), jnp.float32),
        ],
    )(input, w12, b1[None, :], v12, b2[None, :],
      wz, b3[None, :], codebook, cb_hi)

    out = _sc_lookup(cbt, codes.reshape(_B, _S))

    loss_s = loss[0, 0] / jnp.float32(_B * _S * _D)
    return out, loss_s, loss_s
